# baseline (speedup 1.0000x reference)
import functools

import jax
import jax.numpy as jnp
from jax import lax
from jax.experimental import pallas as pl
from jax.experimental.pallas import tpu as pltpu
from jax.experimental.pallas import tpu_sc as plsc

_NC = 2
_NS = 16
_NW = _NC * _NS
_LANES = 16
_MINOR = 128
_ROWQ = 32
_FQ = 512


def _rows_per_worker(total_bytes: int) -> int:
    per = -(-total_bytes // (_NW * _MINOR))
    return -(-per // _ROWQ) * _ROWQ


def _f32_chunks(n: int):
    per = -(-n // _NW)
    a = -(-per // _FQ) * _FQ
    last = n - a * (_NW - 1)
    assert 0 < last <= a and last % _FQ == 0, (a, last)
    return a, last


@functools.lru_cache(maxsize=None)
def _sc_kernel(rows: int, n0: int):
    total_rows = rows * _NW
    a0, last0 = _f32_chunks(n0)
    mesh = plsc.VectorSubcoreMesh(core_axis_name="c", subcore_axis_name="s")

    @functools.partial(
        pl.kernel,
        mesh=mesh,
        out_type=(
            jax.ShapeDtypeStruct((n0,), jnp.float32),
            jax.ShapeDtypeStruct((total_rows, _MINOR), jnp.uint8),
        ),
        scratch_types=[
            pltpu.VMEM((rows, _MINOR), jnp.uint8),
            pltpu.VMEM((rows, _MINOR), jnp.uint8),
            pltpu.SemaphoreType.DMA,
        ],
    )
    def body(s0_h, m_hbm, t_hbm, s0_o, out_hbm, m_v, t_v, sem0):
        wid = lax.axis_index("s") * _NC + lax.axis_index("c")
        base = wid * rows

        @pl.when(wid < _NW - 1)
        def _():
            o0 = wid * a0
            pltpu.async_copy(s0_h.at[pl.ds(o0, a0)], s0_o.at[pl.ds(o0, a0)], sem0)

        @pl.when(wid == _NW - 1)
        def _():
            o0 = (_NW - 1) * a0
            pltpu.async_copy(s0_h.at[pl.ds(o0, last0)], s0_o.at[pl.ds(o0, last0)], sem0)

        pltpu.sync_copy(m_hbm.at[pl.ds(base, rows)], m_v)
        pltpu.sync_copy(t_hbm.at[pl.ds(base, rows)], t_v)

        mw = m_v.bitcast(jnp.int32)
        tw = t_v.bitcast(jnp.int32)

        def step(r, carry):
            for c in range(_MINOR // _LANES):
                sl = pl.ds(c * _LANES, _LANES)
                mw[r, sl] = mw[r, sl] & ~tw[r, sl]
            return carry

        lax.fori_loop(0, rows // 4, step, 0)
        pltpu.sync_copy(m_v, out_hbm.at[pl.ds(base, rows)])

        @pl.when(wid < _NW - 1)
        def _():
            pltpu.make_async_copy(
                s0_h.at[pl.ds(0, a0)], s0_o.at[pl.ds(0, a0)], sem0).wait()

        @pl.when(wid == _NW - 1)
        def _():
            pltpu.make_async_copy(
                s0_h.at[pl.ds(0, last0)], s0_o.at[pl.ds(0, last0)], sem0).wait()

    return body


def kernel(s0, s1, s2, mask, track_mask):
    n = mask.shape[0]
    rows = _rows_per_worker(n)
    total = rows * _NW * _MINOR
    n0 = s0.size

    m = jnp.pad(mask.view(jnp.uint8), (0, total - n)).reshape(rows * _NW, _MINOR)
    t = jnp.pad(track_mask.view(jnp.uint8), (0, total - n)).reshape(rows * _NW, _MINOR)

    s0o, out = _sc_kernel(rows, n0)(s0.reshape(n0), m, t)
    return (
        s0o.reshape(s0.shape),
        s1,
        s2,
        out.reshape(total)[:n].view(jnp.bool_),
    )


# speedup vs baseline: 39.6212x; 39.6212x over previous
import functools

import jax
import jax.numpy as jnp
from jax import lax
from jax.experimental import pallas as pl
from jax.experimental.pallas import tpu as pltpu
from jax.experimental.pallas import tpu_sc as plsc

_NC = 2
_NS = 16
_NW = _NC * _NS
_LANES = 16
_MINOR = 128
_ROWQ = 32


def _rows_per_worker(total_bytes: int) -> int:
    per = -(-total_bytes // (_NW * _MINOR))
    return -(-per // _ROWQ) * _ROWQ


@functools.lru_cache(maxsize=None)
def _sc_mask_kernel(rows: int):
    total_rows = rows * _NW
    mesh = plsc.VectorSubcoreMesh(core_axis_name="c", subcore_axis_name="s")

    @functools.partial(
        pl.kernel,
        mesh=mesh,
        out_type=jax.ShapeDtypeStruct((total_rows, _MINOR), jnp.uint8),
        scratch_types=[
            pltpu.VMEM((rows, _MINOR), jnp.uint8),
            pltpu.VMEM((rows, _MINOR), jnp.uint8),
            pltpu.SemaphoreType.DMA,
            pltpu.SemaphoreType.DMA,
        ],
    )
    def body(m_hbm, t_hbm, out_hbm, m_v, t_v, sem_m, sem_t):
        wid = lax.axis_index("s") * _NC + lax.axis_index("c")
        base = wid * rows
        cm = pltpu.make_async_copy(m_hbm.at[pl.ds(base, rows)], m_v, sem_m)
        ct = pltpu.make_async_copy(t_hbm.at[pl.ds(base, rows)], t_v, sem_t)
        cm.start()
        ct.start()
        cm.wait()
        ct.wait()

        mw = m_v.bitcast(jnp.int32)
        tw = t_v.bitcast(jnp.int32)

        def step(r, carry):
            for c in range(_MINOR // _LANES):
                sl = pl.ds(c * _LANES, _LANES)
                mw[r, sl] = mw[r, sl] & ~tw[r, sl]
            return carry

        lax.fori_loop(0, rows // 4, step, 0)
        pltpu.sync_copy(m_v, out_hbm.at[pl.ds(base, rows)])

    return body


def kernel(s0, s1, s2, mask, track_mask):
    n = mask.shape[0]
    rows = _rows_per_worker(n)
    total = rows * _NW * _MINOR

    m = jnp.pad(mask.view(jnp.uint8), (0, total - n)).reshape(rows * _NW, _MINOR)
    t = jnp.pad(track_mask.view(jnp.uint8), (0, total - n)).reshape(rows * _NW, _MINOR)
    out = _sc_mask_kernel(rows)(m, t)
    return (s0, s1, s2, out.reshape(total)[:n].view(jnp.bool_))


# speedup vs baseline: 39.7092x; 1.0022x over previous
import functools

import jax
import jax.numpy as jnp
from jax import lax
from jax.experimental import pallas as pl
from jax.experimental.pallas import tpu as pltpu
from jax.experimental.pallas import tpu_sc as plsc

_NC = 2
_NS = 16
_NW = _NC * _NS
_LANES = 16
_MINOR = 128
_ROWQ = 32


def _rows_per_worker(total_bytes: int) -> int:
    per = -(-total_bytes // (_NW * _MINOR))
    return -(-per // _ROWQ) * _ROWQ


@functools.lru_cache(maxsize=None)
def _sc_mask_kernel(rows: int):
    total_rows = rows * _NW
    mesh = plsc.VectorSubcoreMesh(core_axis_name="c", subcore_axis_name="s")

    @functools.partial(
        pl.kernel,
        mesh=mesh,
        out_type=jax.ShapeDtypeStruct((total_rows, _MINOR), jnp.uint8),
        scratch_types=[
            pltpu.VMEM((rows, _MINOR), jnp.uint8),
            pltpu.VMEM((rows, _MINOR), jnp.uint8),
            pltpu.SemaphoreType.DMA,
            pltpu.SemaphoreType.DMA,
            pltpu.SemaphoreType.DMA,
        ],
    )
    def body(m_hbm, t_hbm, out_hbm, m_v, t_v, sem_m, sem_t, sem_o):
        wid = lax.axis_index("s") * _NC + lax.axis_index("c")
        base = wid * rows
        half = rows // 2

        def in_copies(h):
            lo = h * half
            cm = pltpu.make_async_copy(
                m_hbm.at[pl.ds(base + lo, half)], m_v.at[pl.ds(lo, half)], sem_m)
            ct = pltpu.make_async_copy(
                t_hbm.at[pl.ds(base + lo, half)], t_v.at[pl.ds(lo, half)], sem_t)
            return cm, ct

        cm0, ct0 = in_copies(0)
        cm1, ct1 = in_copies(1)
        cm0.start()
        ct0.start()
        cm1.start()
        ct1.start()

        mw = m_v.bitcast(jnp.int32)
        tw = t_v.bitcast(jnp.int32)

        def step(r, carry):
            for c in range(_MINOR // _LANES):
                sl = pl.ds(c * _LANES, _LANES)
                mw[r, sl] = mw[r, sl] & ~tw[r, sl]
            return carry

        out_copies = []
        for h in range(2):
            lo = h * half
            cm, ct = (cm0, ct0) if h == 0 else (cm1, ct1)
            cm.wait()
            ct.wait()
            lax.fori_loop(lo // 4, (lo + half) // 4, step, 0)
            co = pltpu.make_async_copy(
                m_v.at[pl.ds(lo, half)], out_hbm.at[pl.ds(base + lo, half)], sem_o)
            co.start()
            out_copies.append(co)
        for co in out_copies:
            co.wait()

    return body


def kernel(s0, s1, s2, mask, track_mask):
    n = mask.shape[0]
    rows = _rows_per_worker(n)
    total = rows * _NW * _MINOR

    m = jnp.pad(mask.view(jnp.uint8), (0, total - n)).reshape(rows * _NW, _MINOR)
    t = jnp.pad(track_mask.view(jnp.uint8), (0, total - n)).reshape(rows * _NW, _MINOR)
    out = _sc_mask_kernel(rows)(m, t)
    return (s0, s1, s2, out.reshape(total)[:n].view(jnp.bool_))


# speedup vs baseline: 39.7312x; 1.0006x over previous
import functools

import jax
import jax.numpy as jnp
from jax import lax
from jax.experimental import pallas as pl
from jax.experimental.pallas import tpu as pltpu
from jax.experimental.pallas import tpu_sc as plsc

_NC = 2
_NS = 16
_NW = _NC * _NS
_LANES = 16
_MINOR = 128
_ROWQ = 32


def _rows_per_worker(total_bytes: int) -> int:
    per = -(-total_bytes // (_NW * _MINOR))
    return -(-per // _ROWQ) * _ROWQ


@functools.lru_cache(maxsize=None)
def _sc_mask_kernel(rows: int):
    total_rows = rows * _NW
    mesh = plsc.VectorSubcoreMesh(core_axis_name="c", subcore_axis_name="s")

    @functools.partial(
        pl.kernel,
        mesh=mesh,
        out_type=jax.ShapeDtypeStruct((total_rows, _MINOR), jnp.uint8),
        scratch_types=[
            pltpu.VMEM((rows, _MINOR), jnp.uint8),
            pltpu.VMEM((rows, _MINOR), jnp.uint8),
            pltpu.SemaphoreType.DMA,
            pltpu.SemaphoreType.DMA,
            pltpu.SemaphoreType.DMA,
        ],
    )
    def body(m_hbm, t_hbm, out_hbm, m_v, t_v, sem_m, sem_t, sem_o):
        wid = lax.axis_index("s") * _NC + lax.axis_index("c")
        base = wid * rows
        half = rows // 2

        def in_copies(h):
            lo = h * half
            cm = pltpu.make_async_copy(
                m_hbm.at[pl.ds(base + lo, half)], m_v.at[pl.ds(lo, half)], sem_m)
            ct = pltpu.make_async_copy(
                t_hbm.at[pl.ds(base + lo, half)], t_v.at[pl.ds(lo, half)], sem_t)
            return cm, ct

        cm0, ct0 = in_copies(0)
        cm1, ct1 = in_copies(1)
        cm0.start()
        ct0.start()
        cm1.start()
        ct1.start()

        mw = m_v.bitcast(jnp.int32)
        tw = t_v.bitcast(jnp.int32)

        def step(r, carry):
            for c in range(_MINOR // _LANES):
                sl = pl.ds(c * _LANES, _LANES)
                mw[r, sl] = mw[r, sl] & ~tw[r, sl]
            return carry

        out_copies = []
        for h in range(2):
            lo = h * half
            cm, ct = (cm0, ct0) if h == 0 else (cm1, ct1)
            cm.wait()
            ct.wait()
            lax.fori_loop(lo // 4, (lo + half) // 4, step, 0)
            co = pltpu.make_async_copy(
                m_v.at[pl.ds(lo, half)], out_hbm.at[pl.ds(base + lo, half)], sem_o)
            co.start()
            out_copies.append(co)
        for co in out_copies:
            co.wait()

    return body


def kernel(s0, s1, s2, mask, track_mask):
    n = mask.shape[0]
    rows = _rows_per_worker(n)
    total = rows * _NW * _MINOR

    m = jnp.pad(mask.view(jnp.uint8), (0, total - n)).reshape(rows * _NW, _MINOR)
    t = jnp.pad(track_mask.view(jnp.uint8), (0, total - n)).reshape(rows * _NW, _MINOR)
    out = _sc_mask_kernel(rows)(m, t)
    return (s0 * jnp.float32(1.0), s1, s2, out.reshape(total)[:n].view(jnp.bool_))
